# baseline (speedup 1.0000x reference)
import functools

import jax
import jax.numpy as jnp
from jax.experimental import pallas as pl
from jax.experimental.pallas import tpu as pltpu

_BF = jnp.bfloat16


def _fold_bn_base(w, gamma, beta, mean, var, wdtype=_BF):
    cout, cin, kh, kw = w.shape
    scale = gamma / jnp.sqrt(var + 1e-5)
    wm = jnp.transpose(w, (2, 3, 1, 0)).reshape(kh * kw * cin, cout)
    wm = wm * scale[None, :]
    bias = (beta - mean * scale).reshape(1, cout)
    return wm.astype(_BF).astype(wdtype), bias.astype(jnp.float32)


def _conv3x3(src, w_ref, b_ref, oh, ow, stride, res=None, sf_ref=None,
             pp_ref=None):
    c = src.shape[-1]
    taps = []
    for i in range(3):
        for j in range(3):
            if stride == 1:
                taps.append(src[:, i:i + oh, j:j + ow, :])
            else:
                vv = src[:, i:i + 2 * oh, j:j + 2 * ow, :]
                taps.append(vv.reshape(vv.shape[0], oh, 2, 2 * ow, c)[:, :, 0])
    p = jnp.concatenate(taps, axis=-1)
    ib = p.shape[0]
    m = ib * oh * ow
    cout = w_ref.shape[-1]
    p = p.reshape(ib * p.shape[1] * p.shape[2], p.shape[-1])
    acc = jnp.dot(p, w_ref[...], preferred_element_type=jnp.float32)
    acc = acc + b_ref[...]
    if stride == 2:
        accr = acc.reshape(ib, oh, 2 * ow, cout)
        subs = []
        for g in range(cout // 128):
            sf_ref[g] = accr[..., g * 128:(g + 1) * 128]
            subs.append(sf_ref[g, :, :, pl.ds(0, ow, 2), :])
        acc = jnp.concatenate(subs, axis=-1).reshape(m, cout)
    if res is not None:
        acc = acc + res.astype(jnp.float32)
    return jnp.maximum(acc, 0.0).astype(_BF)


def _store_padded(dst_ref, val, ib, oh, ow, c):
    dst_ref[...] = jnp.zeros(dst_ref.shape, dst_ref.dtype)
    dst_ref[:, 1:1 + oh, 1:1 + ow, :] = (
        val.reshape(ib, oh, ow, c).astype(dst_ref.dtype))


def _stem_kernel(x_ref, w_ref, b_ref, o_ref, sp_ref, *, oh1):
    ib = x_ref.shape[0]
    sp_ref[...] = jnp.zeros(sp_ref.shape, sp_ref.dtype)
    nch = max(1, oh1 // 28)
    ch = oh1 // nch
    for k in range(nch):
        r0 = k * ch
        acc = None
        for p in range(4):
            v = x_ref[:, r0 + p:r0 + p + ch, :, :]
            d = jnp.dot(v.reshape(ib * ch * oh1, 64),
                        w_ref[p * 64:(p + 1) * 64, :],
                        preferred_element_type=jnp.float32)
            acc = d if acc is None else acc + d
        y = jnp.maximum(acc + b_ref[...], 0.0)
        sp_ref[:, 1 + r0:1 + r0 + ch, 1:1 + oh1, 0:64] = (
            y.reshape(ib, ch, oh1, 64))
    po = oh1 // 2
    npc = max(1, po // 28)
    pc = po // npc
    o_ref[...] = jnp.zeros(o_ref.shape, o_ref.dtype)
    for k in range(npc):
        c0 = k * pc
        mts = []
        for i in range(3):
            for j in range(3):
                v = sp_ref[:, pl.ds(2 * c0 + i, 2 * pc), pl.ds(j, po, 2), :]
                mts.append(v.reshape(ib, pc, 2, po, 128)[:, :, 0])
        pooled = functools.reduce(jnp.maximum, mts)
        o_ref[:, 1 + c0:1 + c0 + pc, 1:1 + po, :] = (
            pooled[..., 0:64].astype(_BF).astype(o_ref.dtype))


def _layer_kernel(*refs, hin, cout, stride, has_down, gap_out):
    it = iter(refs)
    x_ref = next(it)
    w1, b1 = next(it), next(it)
    if has_down:
        wd, bd = next(it), next(it)
    w2, b2 = next(it), next(it)
    w3, b3 = next(it), next(it)
    w4, b4 = next(it), next(it)
    o_ref = next(it)
    sa_ref, sb_ref = next(it), next(it)
    sf_ref = next(it) if stride == 2 else None

    ib = x_ref.shape[0]
    oh = hin // stride
    m = ib * oh * oh

    h1 = _conv3x3(x_ref, w1, b1, oh, oh, stride, sf_ref=sf_ref)
    _store_padded(sa_ref, h1, ib, oh, oh, cout)

    if has_down:
        v = x_ref[:, 1:1 + 2 * oh, 1:1 + 2 * oh, :]
        cin = v.shape[-1]
        xd = v.reshape(ib, oh, 2, 2 * oh, cin)[:, :, 0]
        xd = xd.reshape(ib * oh * 2 * oh, cin)
        dacc = jnp.dot(xd, wd[...], preferred_element_type=jnp.float32)
        daccr = (dacc + bd[...]).reshape(ib, oh, 2 * oh, cout)
        subs = []
        for g in range(cout // 128):
            sf_ref[g] = daccr[..., g * 128:(g + 1) * 128]
            subs.append(sf_ref[g, :, :, pl.ds(0, oh, 2), :])
        idn = jnp.concatenate(subs, axis=-1).reshape(m, cout).astype(_BF)
    else:
        idn = x_ref[:, 1:1 + oh, 1:1 + oh, :].reshape(m, cout)

    h2 = _conv3x3(sa_ref, w2, b2, oh, oh, 1, res=idn)
    _store_padded(sb_ref, h2, ib, oh, oh, cout)

    h3 = _conv3x3(sb_ref, w3, b3, oh, oh, 1)
    _store_padded(sa_ref, h3, ib, oh, oh, cout)

    h4 = _conv3x3(sa_ref, w4, b4, oh, oh, 1, res=h2)

    if gap_out:
        f = h4.reshape(ib, oh * oh, cout).astype(jnp.float32)
        o_ref[...] = jnp.mean(f, axis=1).astype(_BF)
    else:
        o_ref[...] = jnp.zeros(o_ref.shape, o_ref.dtype)
        o_ref[:, 1:1 + oh, 1:1 + oh, :] = h4.reshape(ib, oh, oh, cout)


def _fc_kernel(a_ref, w_ref, b_ref, o_ref):
    acc = jnp.dot(a_ref[...], w_ref[...], preferred_element_type=jnp.float32)
    o_ref[...] = acc + b_ref[...]


_NCORES = 1


def _cparams():
    return pltpu.CompilerParams(
        dimension_semantics=("arbitrary", "arbitrary"),
        vmem_limit_bytes=48 * 1024 * 1024)


def _split(n):
    nc = _NCORES if n % _NCORES == 0 else 1
    if nc == 1:
        return (1, n), n
    return (nc, n // nc), n // nc


def _run_stem(xq, w2, bias, ib):
    b, hs, ow1, _ = xq.shape
    oh1 = hs - 4
    po = oh1 // 2
    grid, spc = _split(b // ib)
    return pl.pallas_call(
        functools.partial(_stem_kernel, oh1=oh1),
        out_shape=jax.ShapeDtypeStruct((b, po + 2, po + 2, 64), jnp.float32),
        grid=grid,
        in_specs=[
            pl.BlockSpec((ib, hs, ow1, 64),
                         lambda c, i, s=spc: (c * s + i, 0, 0, 0)),
            pl.BlockSpec(w2.shape, lambda c, i: (0, 0)),
            pl.BlockSpec(bias.shape, lambda c, i: (0, 0)),
        ],
        out_specs=pl.BlockSpec((ib, po + 2, po + 2, 64),
                               lambda c, i, s=spc: (c * s + i, 0, 0, 0)),
        scratch_shapes=[
            pltpu.VMEM((ib, oh1 + 2, oh1 + 2, 128), jnp.float32),
        ],
        compiler_params=_cparams(),
    )(xq, w2, bias)


def _run_layer(x, weights, *, cout, stride, has_down, gap_out, ib,
               act32=False):
    b, hp, _, cin = x.shape
    hin = hp - 2
    oh = hin // stride
    grid, spc = _split(b // ib)
    if gap_out:
        out_shape = jax.ShapeDtypeStruct((b, cout), _BF)
        out_spec = pl.BlockSpec((ib, cout), lambda c, i, s=spc: (c * s + i, 0))
    else:
        out_shape = jax.ShapeDtypeStruct((b, oh + 2, oh + 2, cout), _BF)
        out_spec = pl.BlockSpec((ib, oh + 2, oh + 2, cout),
                                lambda c, i, s=spc: (c * s + i, 0, 0, 0))
    in_specs = [pl.BlockSpec((ib, hp, hp, cin),
                             lambda c, i, s=spc: (c * s + i, 0, 0, 0))]
    for wb in weights:
        in_specs.append(pl.BlockSpec(wb.shape, lambda c, i: (0, 0)))
    adt = jnp.float32 if act32 else _BF
    scratch = [pltpu.VMEM((ib, oh + 2, oh + 2, cout), adt),
               pltpu.VMEM((ib, oh + 2, oh + 2, cout), adt)]
    if stride == 2:
        scratch.append(
            pltpu.VMEM((cout // 128, ib, oh, 2 * oh, 128), jnp.float32))
    return pl.pallas_call(
        functools.partial(_layer_kernel, hin=hin, cout=cout, stride=stride,
                          has_down=has_down, gap_out=gap_out),
        out_shape=out_shape,
        grid=grid,
        in_specs=in_specs,
        out_specs=out_spec,
        scratch_shapes=scratch,
        compiler_params=_cparams(),
    )(x, *weights)


def _stem_prep(conv1_w, gamma, beta, mean, var, x):
    b, _, h, w = x.shape
    xp = jnp.pad(x, ((0, 0), (0, 0), (3, 5), (3, 5)))
    hs, ws = (h + 8) // 2, (w + 8) // 2
    ow1 = ws - 4
    parts = []
    for q in range(4):
        for a in range(2):
            for bb in range(2):
                for c in range(3):
                    sl = jax.lax.slice(
                        xp, (0, c, a, 2 * q + bb),
                        (b, c + 1, a + 2 * (hs - 1) + 1,
                         2 * q + bb + 2 * (ow1 - 1) + 1),
                        (1, 1, 2, 2))
                    parts.append(sl.reshape(b, hs, ow1))
    xq = jnp.stack(parts, axis=-1).astype(_BF)
    xq = jnp.pad(xq, ((0, 0), (0, 0), (0, 0), (0, 16)))

    scale = gamma / jnp.sqrt(var + 1e-5)
    wp = jnp.pad(conv1_w, ((0, 0), (0, 0), (0, 1), (0, 1)))
    wt = wp.reshape(64, 3, 4, 2, 4, 2).transpose(2, 4, 3, 5, 1, 0)
    w2 = wt.reshape(192, 64) * scale[None, :]
    w2 = jnp.pad(w2.reshape(4, 48, 64), ((0, 0), (0, 16), (0, 0)))
    w2 = w2.reshape(256, 64)
    bias = (beta - mean * scale).reshape(1, 64)
    return xq, w2.astype(_BF), bias.astype(jnp.float32)


def kernel(conv1_w, bn1_gamma, bn1_beta, bn1_mean, bn1_var, fc_w, fc_b, l1_b0_conv1_w, l1_b0_bn1_gamma, l1_b0_bn1_beta, l1_b0_bn1_mean, l1_b0_bn1_var, l1_b0_conv2_w, l1_b0_bn2_gamma, l1_b0_bn2_beta, l1_b0_bn2_mean, l1_b0_bn2_var, l1_b1_conv1_w, l1_b1_bn1_gamma, l1_b1_bn1_beta, l1_b1_bn1_mean, l1_b1_bn1_var, l1_b1_conv2_w, l1_b1_bn2_gamma, l1_b1_bn2_beta, l1_b1_bn2_mean, l1_b1_bn2_var, l2_b0_conv1_w, l2_b0_bn1_gamma, l2_b0_bn1_beta, l2_b0_bn1_mean, l2_b0_bn1_var, l2_b0_conv2_w, l2_b0_bn2_gamma, l2_b0_bn2_beta, l2_b0_bn2_mean, l2_b0_bn2_var, l2_b0_down_w, l2_b0_down_bn_gamma, l2_b0_down_bn_beta, l2_b0_down_bn_mean, l2_b0_down_bn_var, l2_b1_conv1_w, l2_b1_bn1_gamma, l2_b1_bn1_beta, l2_b1_bn1_mean, l2_b1_bn1_var, l2_b1_conv2_w, l2_b1_bn2_gamma, l2_b1_bn2_beta, l2_b1_bn2_mean, l2_b1_bn2_var, l3_b0_conv1_w, l3_b0_bn1_gamma, l3_b0_bn1_beta, l3_b0_bn1_mean, l3_b0_bn1_var, l3_b0_conv2_w, l3_b0_bn2_gamma, l3_b0_bn2_beta, l3_b0_bn2_mean, l3_b0_bn2_var, l3_b0_down_w, l3_b0_down_bn_gamma, l3_b0_down_bn_beta, l3_b0_down_bn_mean, l3_b0_down_bn_var, l3_b1_conv1_w, l3_b1_bn1_gamma, l3_b1_bn1_beta, l3_b1_bn1_mean, l3_b1_bn1_var, l3_b1_conv2_w, l3_b1_bn2_gamma, l3_b1_bn2_beta, l3_b1_bn2_mean, l3_b1_bn2_var, l4_b0_conv1_w, l4_b0_bn1_gamma, l4_b0_bn1_beta, l4_b0_bn1_mean, l4_b0_bn1_var, l4_b0_conv2_w, l4_b0_bn2_gamma, l4_b0_bn2_beta, l4_b0_bn2_mean, l4_b0_bn2_var, l4_b0_down_w, l4_b0_down_bn_gamma, l4_b0_down_bn_beta, l4_b0_down_bn_mean, l4_b0_down_bn_var, l4_b1_conv1_w, l4_b1_bn1_gamma, l4_b1_bn1_beta, l4_b1_bn1_mean, l4_b1_bn1_var, l4_b1_conv2_w, l4_b1_bn2_gamma, l4_b1_bn2_beta, l4_b1_bn2_mean, l4_b1_bn2_var, x):
    b = x.shape[0]

    xs, w2, bias1 = _stem_prep(conv1_w, bn1_gamma, bn1_beta, bn1_mean,
                               bn1_var, x)
    h = _run_stem(xs, w2, bias1, ib=1)

    layer_args = [
        ('l1', 64, 1, False, 1, True),
        ('l2', 128, 2, True, 1, False),
        ('l3', 256, 2, True, 2 if b % 2 == 0 else 1, False),
        ('l4', 512, 2, True, 8 if b % 8 == 0 else 1, False),
    ]
    raw = locals()
    for name, cout, stride, has_down, ib, act32 in layer_args:
        wdt = jnp.float32 if act32 else _BF
        _fold_bn = functools.partial(_fold_bn_base, wdtype=wdt)
        weights = []
        weights += _fold_bn(raw[f'{name}_b0_conv1_w'],
                            raw[f'{name}_b0_bn1_gamma'],
                            raw[f'{name}_b0_bn1_beta'],
                            raw[f'{name}_b0_bn1_mean'],
                            raw[f'{name}_b0_bn1_var'])
        if has_down:
            weights += _fold_bn(raw[f'{name}_b0_down_w'],
                                raw[f'{name}_b0_down_bn_gamma'],
                                raw[f'{name}_b0_down_bn_beta'],
                                raw[f'{name}_b0_down_bn_mean'],
                                raw[f'{name}_b0_down_bn_var'])
        weights += _fold_bn(raw[f'{name}_b0_conv2_w'],
                            raw[f'{name}_b0_bn2_gamma'],
                            raw[f'{name}_b0_bn2_beta'],
                            raw[f'{name}_b0_bn2_mean'],
                            raw[f'{name}_b0_bn2_var'])
        weights += _fold_bn(raw[f'{name}_b1_conv1_w'],
                            raw[f'{name}_b1_bn1_gamma'],
                            raw[f'{name}_b1_bn1_beta'],
                            raw[f'{name}_b1_bn1_mean'],
                            raw[f'{name}_b1_bn1_var'])
        weights += _fold_bn(raw[f'{name}_b1_conv2_w'],
                            raw[f'{name}_b1_bn2_gamma'],
                            raw[f'{name}_b1_bn2_beta'],
                            raw[f'{name}_b1_bn2_mean'],
                            raw[f'{name}_b1_bn2_var'])
        h = _run_layer(h, weights, cout=cout, stride=stride,
                       has_down=has_down, gap_out=(name == 'l4'), ib=ib,
                       act32=act32)

    ncls = fc_w.shape[0]
    np_ = (ncls + 127) // 128 * 128
    fw = jnp.pad(fc_w.T.astype(_BF), ((0, 0), (0, np_ - ncls)))
    fb = jnp.pad(fc_b, (0, np_ - ncls)).reshape(1, np_).astype(jnp.float32)
    logits = pl.pallas_call(
        _fc_kernel,
        out_shape=jax.ShapeDtypeStruct((b, np_), jnp.float32),
        grid=(_NCORES, 1),
        in_specs=[pl.BlockSpec((b // _NCORES, 512),
                               lambda c, i: (c, 0)),
                  pl.BlockSpec((512, np_), lambda c, i: (0, 0)),
                  pl.BlockSpec((1, np_), lambda c, i: (0, 0))],
        out_specs=pl.BlockSpec((b // _NCORES, np_), lambda c, i: (c, 0)),
        compiler_params=_cparams(),
    )(h, fw, fb)
    return logits[:, :ncls]


# speedup vs baseline: 8.5665x; 8.5665x over previous
import functools

import jax
import jax.numpy as jnp
from jax.experimental import pallas as pl
from jax.experimental.pallas import tpu as pltpu

_BF = jnp.bfloat16


def _fold_bn_base(w, gamma, beta, mean, var, wdtype=_BF):
    cout, cin, kh, kw = w.shape
    scale = gamma / jnp.sqrt(var + 1e-5)
    wm = jnp.transpose(w, (2, 3, 1, 0)).reshape(kh * kw * cin, cout)
    wm = wm * scale[None, :]
    bias = (beta - mean * scale).reshape(1, cout)
    return wm.astype(_BF).astype(wdtype), bias.astype(jnp.float32)


def _conv3x3(src, w_ref, b_ref, oh, ow, stride, res=None, sf_ref=None,
             pp_ref=None):
    c = src.shape[-1]
    taps = []
    for i in range(3):
        for j in range(3):
            if stride == 1:
                taps.append(src[:, i:i + oh, j:j + ow, :])
            else:
                vv = src[:, i:i + 2 * oh, j:j + 2 * ow, :]
                taps.append(vv.reshape(vv.shape[0], oh, 2, 2 * ow, c)[:, :, 0])
    p = jnp.concatenate(taps, axis=-1)
    ib = p.shape[0]
    m = ib * oh * ow
    cout = w_ref.shape[-1]
    p = p.reshape(ib * p.shape[1] * p.shape[2], p.shape[-1])
    acc = jnp.dot(p, w_ref[...], preferred_element_type=jnp.float32)
    acc = acc + b_ref[...]
    if stride == 2:
        accr = acc.reshape(ib, oh, 2 * ow, cout)
        subs = []
        for g in range(cout // 128):
            sf_ref[g] = accr[..., g * 128:(g + 1) * 128]
            subs.append(sf_ref[g, :, :, pl.ds(0, ow, 2), :])
        acc = jnp.concatenate(subs, axis=-1).reshape(m, cout)
    if res is not None:
        acc = acc + res.astype(jnp.float32)
    return jnp.maximum(acc, 0.0).astype(_BF)


def _store_padded(dst_ref, val, ib, oh, ow, c):
    dst_ref[...] = jnp.zeros(dst_ref.shape, dst_ref.dtype)
    dst_ref[:, 1:1 + oh, 1:1 + ow, :] = (
        val.reshape(ib, oh, ow, c).astype(dst_ref.dtype))


def _stem_kernel(x_ref, w_ref, b_ref, o_ref, sp_ref, *, oh1):
    ib = x_ref.shape[0]
    sp_ref[...] = jnp.zeros(sp_ref.shape, sp_ref.dtype)
    nch = max(1, oh1 // 28)
    ch = oh1 // nch
    for k in range(nch):
        r0 = k * ch
        taps = [x_ref[:, r0 + p:r0 + p + ch, q:q + oh1, :]
                for p in range(4) for q in range(4)]
        pm = jnp.concatenate(taps, axis=-1).reshape(ib * ch * oh1, 16 * 12)
        acc = jnp.dot(pm, w_ref[...], preferred_element_type=jnp.float32)
        y = jnp.maximum(acc + b_ref[...], 0.0)
        sp_ref[:, 1 + r0:1 + r0 + ch, 1:1 + oh1, 0:64] = (
            y.reshape(ib, ch, oh1, 64))
    po = oh1 // 2
    npc = max(1, po // 28)
    pc = po // npc
    o_ref[...] = jnp.zeros(o_ref.shape, o_ref.dtype)
    for k in range(npc):
        c0 = k * pc
        mts = []
        for i in range(3):
            for j in range(3):
                v = sp_ref[:, pl.ds(2 * c0 + i, 2 * pc), pl.ds(j, po, 2), :]
                mts.append(v.reshape(ib, pc, 2, po, 128)[:, :, 0])
        pooled = functools.reduce(jnp.maximum, mts)
        o_ref[:, 1 + c0:1 + c0 + pc, 1:1 + po, :] = (
            pooled[..., 0:64].astype(_BF).astype(o_ref.dtype))


def _layer_kernel(*refs, hin, cout, stride, has_down, gap_out):
    it = iter(refs)
    x_ref = next(it)
    w1, b1 = next(it), next(it)
    if has_down:
        wd, bd = next(it), next(it)
    w2, b2 = next(it), next(it)
    w3, b3 = next(it), next(it)
    w4, b4 = next(it), next(it)
    o_ref = next(it)
    sa_ref, sb_ref = next(it), next(it)
    sf_ref = next(it) if stride == 2 else None

    ib = x_ref.shape[0]
    oh = hin // stride
    m = ib * oh * oh

    h1 = _conv3x3(x_ref, w1, b1, oh, oh, stride, sf_ref=sf_ref)
    _store_padded(sa_ref, h1, ib, oh, oh, cout)

    if has_down:
        v = x_ref[:, 1:1 + 2 * oh, 1:1 + 2 * oh, :]
        cin = v.shape[-1]
        xd = v.reshape(ib, oh, 2, 2 * oh, cin)[:, :, 0]
        xd = xd.reshape(ib * oh * 2 * oh, cin)
        dacc = jnp.dot(xd, wd[...], preferred_element_type=jnp.float32)
        daccr = (dacc + bd[...]).reshape(ib, oh, 2 * oh, cout)
        subs = []
        for g in range(cout // 128):
            sf_ref[g] = daccr[..., g * 128:(g + 1) * 128]
            subs.append(sf_ref[g, :, :, pl.ds(0, oh, 2), :])
        idn = jnp.concatenate(subs, axis=-1).reshape(m, cout).astype(_BF)
    else:
        idn = x_ref[:, 1:1 + oh, 1:1 + oh, :].reshape(m, cout)

    h2 = _conv3x3(sa_ref, w2, b2, oh, oh, 1, res=idn)
    _store_padded(sb_ref, h2, ib, oh, oh, cout)

    h3 = _conv3x3(sb_ref, w3, b3, oh, oh, 1)
    _store_padded(sa_ref, h3, ib, oh, oh, cout)

    h4 = _conv3x3(sa_ref, w4, b4, oh, oh, 1, res=h2)

    if gap_out:
        f = h4.reshape(ib, oh * oh, cout).astype(jnp.float32)
        o_ref[...] = jnp.mean(f, axis=1).astype(_BF)
    else:
        o_ref[...] = jnp.zeros(o_ref.shape, o_ref.dtype)
        o_ref[:, 1:1 + oh, 1:1 + oh, :] = h4.reshape(ib, oh, oh, cout)


def _fc_kernel(a_ref, w_ref, b_ref, o_ref):
    acc = jnp.dot(a_ref[...], w_ref[...], preferred_element_type=jnp.float32)
    o_ref[...] = acc + b_ref[...]


_NCORES = 1


def _cparams():
    return pltpu.CompilerParams(
        dimension_semantics=("arbitrary", "arbitrary"),
        vmem_limit_bytes=48 * 1024 * 1024)


def _split(n):
    nc = _NCORES if n % _NCORES == 0 else 1
    if nc == 1:
        return (1, n), n
    return (nc, n // nc), n // nc


def _run_stem(xs, w2, bias, ib):
    b, hs, _, _ = xs.shape
    oh1 = hs - 4
    po = oh1 // 2
    grid, spc = _split(b // ib)
    return pl.pallas_call(
        functools.partial(_stem_kernel, oh1=oh1),
        out_shape=jax.ShapeDtypeStruct((b, po + 2, po + 2, 64), _BF),
        grid=grid,
        in_specs=[
            pl.BlockSpec((ib, hs, hs, 12),
                         lambda c, i, s=spc: (c * s + i, 0, 0, 0)),
            pl.BlockSpec(w2.shape, lambda c, i: (0, 0)),
            pl.BlockSpec(bias.shape, lambda c, i: (0, 0)),
        ],
        out_specs=pl.BlockSpec((ib, po + 2, po + 2, 64),
                               lambda c, i, s=spc: (c * s + i, 0, 0, 0)),
        scratch_shapes=[
            pltpu.VMEM((ib, oh1 + 2, oh1 + 2, 128), jnp.float32),
        ],
        compiler_params=_cparams(),
    )(xs, w2, bias)


def _run_layer(x, weights, *, cout, stride, has_down, gap_out, ib,
               act32=False):
    b, hp, _, cin = x.shape
    hin = hp - 2
    oh = hin // stride
    grid, spc = _split(b // ib)
    if gap_out:
        out_shape = jax.ShapeDtypeStruct((b, cout), _BF)
        out_spec = pl.BlockSpec((ib, cout), lambda c, i, s=spc: (c * s + i, 0))
    else:
        out_shape = jax.ShapeDtypeStruct((b, oh + 2, oh + 2, cout), _BF)
        out_spec = pl.BlockSpec((ib, oh + 2, oh + 2, cout),
                                lambda c, i, s=spc: (c * s + i, 0, 0, 0))
    in_specs = [pl.BlockSpec((ib, hp, hp, cin),
                             lambda c, i, s=spc: (c * s + i, 0, 0, 0))]
    for wb in weights:
        in_specs.append(pl.BlockSpec(wb.shape, lambda c, i: (0, 0)))
    adt = jnp.float32 if act32 else _BF
    scratch = [pltpu.VMEM((ib, oh + 2, oh + 2, cout), adt),
               pltpu.VMEM((ib, oh + 2, oh + 2, cout), adt)]
    if stride == 2:
        scratch.append(
            pltpu.VMEM((cout // 128, ib, oh, 2 * oh, 128), jnp.float32))
    return pl.pallas_call(
        functools.partial(_layer_kernel, hin=hin, cout=cout, stride=stride,
                          has_down=has_down, gap_out=gap_out),
        out_shape=out_shape,
        grid=grid,
        in_specs=in_specs,
        out_specs=out_spec,
        scratch_shapes=scratch,
        compiler_params=_cparams(),
    )(x, *weights)


def _stem_prep(conv1_w, gamma, beta, mean, var, x):
    b, _, h, w = x.shape
    xp = jnp.pad(x, ((0, 0), (0, 0), (3, 5), (3, 5)))
    hs, ws = (h + 8) // 2, (w + 8) // 2
    xs = xp.reshape(b, 3, hs, 2, ws, 2)
    xs = xs.transpose(0, 2, 4, 3, 5, 1).reshape(b, hs, ws, 12).astype(_BF)

    scale = gamma / jnp.sqrt(var + 1e-5)
    wp = jnp.pad(conv1_w, ((0, 0), (0, 0), (0, 1), (0, 1)))
    wt = wp.reshape(64, 3, 4, 2, 4, 2).transpose(2, 4, 3, 5, 1, 0)
    w2 = wt.reshape(192, 64) * scale[None, :]
    bias = (beta - mean * scale).reshape(1, 64)
    return xs, w2.astype(_BF), bias.astype(jnp.float32)


def kernel(conv1_w, bn1_gamma, bn1_beta, bn1_mean, bn1_var, fc_w, fc_b, l1_b0_conv1_w, l1_b0_bn1_gamma, l1_b0_bn1_beta, l1_b0_bn1_mean, l1_b0_bn1_var, l1_b0_conv2_w, l1_b0_bn2_gamma, l1_b0_bn2_beta, l1_b0_bn2_mean, l1_b0_bn2_var, l1_b1_conv1_w, l1_b1_bn1_gamma, l1_b1_bn1_beta, l1_b1_bn1_mean, l1_b1_bn1_var, l1_b1_conv2_w, l1_b1_bn2_gamma, l1_b1_bn2_beta, l1_b1_bn2_mean, l1_b1_bn2_var, l2_b0_conv1_w, l2_b0_bn1_gamma, l2_b0_bn1_beta, l2_b0_bn1_mean, l2_b0_bn1_var, l2_b0_conv2_w, l2_b0_bn2_gamma, l2_b0_bn2_beta, l2_b0_bn2_mean, l2_b0_bn2_var, l2_b0_down_w, l2_b0_down_bn_gamma, l2_b0_down_bn_beta, l2_b0_down_bn_mean, l2_b0_down_bn_var, l2_b1_conv1_w, l2_b1_bn1_gamma, l2_b1_bn1_beta, l2_b1_bn1_mean, l2_b1_bn1_var, l2_b1_conv2_w, l2_b1_bn2_gamma, l2_b1_bn2_beta, l2_b1_bn2_mean, l2_b1_bn2_var, l3_b0_conv1_w, l3_b0_bn1_gamma, l3_b0_bn1_beta, l3_b0_bn1_mean, l3_b0_bn1_var, l3_b0_conv2_w, l3_b0_bn2_gamma, l3_b0_bn2_beta, l3_b0_bn2_mean, l3_b0_bn2_var, l3_b0_down_w, l3_b0_down_bn_gamma, l3_b0_down_bn_beta, l3_b0_down_bn_mean, l3_b0_down_bn_var, l3_b1_conv1_w, l3_b1_bn1_gamma, l3_b1_bn1_beta, l3_b1_bn1_mean, l3_b1_bn1_var, l3_b1_conv2_w, l3_b1_bn2_gamma, l3_b1_bn2_beta, l3_b1_bn2_mean, l3_b1_bn2_var, l4_b0_conv1_w, l4_b0_bn1_gamma, l4_b0_bn1_beta, l4_b0_bn1_mean, l4_b0_bn1_var, l4_b0_conv2_w, l4_b0_bn2_gamma, l4_b0_bn2_beta, l4_b0_bn2_mean, l4_b0_bn2_var, l4_b0_down_w, l4_b0_down_bn_gamma, l4_b0_down_bn_beta, l4_b0_down_bn_mean, l4_b0_down_bn_var, l4_b1_conv1_w, l4_b1_bn1_gamma, l4_b1_bn1_beta, l4_b1_bn1_mean, l4_b1_bn1_var, l4_b1_conv2_w, l4_b1_bn2_gamma, l4_b1_bn2_beta, l4_b1_bn2_mean, l4_b1_bn2_var, x):
    b = x.shape[0]

    xs, w2, bias1 = _stem_prep(conv1_w, bn1_gamma, bn1_beta, bn1_mean,
                               bn1_var, x)
    h = _run_stem(xs, w2, bias1, ib=1)

    layer_args = [
        ('l1', 64, 1, False, 1, False),
        ('l2', 128, 2, True, 1, False),
        ('l3', 256, 2, True, 2 if b % 2 == 0 else 1, False),
        ('l4', 512, 2, True, 8 if b % 8 == 0 else 1, False),
    ]
    raw = locals()
    for name, cout, stride, has_down, ib, act32 in layer_args:
        wdt = jnp.float32 if act32 else _BF
        _fold_bn = functools.partial(_fold_bn_base, wdtype=wdt)
        weights = []
        weights += _fold_bn(raw[f'{name}_b0_conv1_w'],
                            raw[f'{name}_b0_bn1_gamma'],
                            raw[f'{name}_b0_bn1_beta'],
                            raw[f'{name}_b0_bn1_mean'],
                            raw[f'{name}_b0_bn1_var'])
        if has_down:
            weights += _fold_bn(raw[f'{name}_b0_down_w'],
                                raw[f'{name}_b0_down_bn_gamma'],
                                raw[f'{name}_b0_down_bn_beta'],
                                raw[f'{name}_b0_down_bn_mean'],
                                raw[f'{name}_b0_down_bn_var'])
        weights += _fold_bn(raw[f'{name}_b0_conv2_w'],
                            raw[f'{name}_b0_bn2_gamma'],
                            raw[f'{name}_b0_bn2_beta'],
                            raw[f'{name}_b0_bn2_mean'],
                            raw[f'{name}_b0_bn2_var'])
        weights += _fold_bn(raw[f'{name}_b1_conv1_w'],
                            raw[f'{name}_b1_bn1_gamma'],
                            raw[f'{name}_b1_bn1_beta'],
                            raw[f'{name}_b1_bn1_mean'],
                            raw[f'{name}_b1_bn1_var'])
        weights += _fold_bn(raw[f'{name}_b1_conv2_w'],
                            raw[f'{name}_b1_bn2_gamma'],
                            raw[f'{name}_b1_bn2_beta'],
                            raw[f'{name}_b1_bn2_mean'],
                            raw[f'{name}_b1_bn2_var'])
        h = _run_layer(h, weights, cout=cout, stride=stride,
                       has_down=has_down, gap_out=(name == 'l4'), ib=ib,
                       act32=act32)

    ncls = fc_w.shape[0]
    np_ = (ncls + 127) // 128 * 128
    fw = jnp.pad(fc_w.T.astype(_BF), ((0, 0), (0, np_ - ncls)))
    fb = jnp.pad(fc_b, (0, np_ - ncls)).reshape(1, np_).astype(jnp.float32)
    logits = pl.pallas_call(
        _fc_kernel,
        out_shape=jax.ShapeDtypeStruct((b, np_), jnp.float32),
        grid=(_NCORES, 1),
        in_specs=[pl.BlockSpec((b // _NCORES, 512),
                               lambda c, i: (c, 0)),
                  pl.BlockSpec((512, np_), lambda c, i: (0, 0)),
                  pl.BlockSpec((1, np_), lambda c, i: (0, 0))],
        out_specs=pl.BlockSpec((b // _NCORES, np_), lambda c, i: (c, 0)),
        compiler_params=_cparams(),
    )(h, fw, fb)
    return logits[:, :ncls]
